# Initial kernel scaffold; baseline (speedup 1.0000x reference)
import jax
import jax.numpy as jnp
from jax.experimental import pallas as pl


def kernel(x):
    raise NotImplementedError("write your pallas kernel here")


# speedup vs baseline: 176.8683x; 176.8683x over previous
import functools

import jax
import jax.numpy as jnp
from jax.experimental import pallas as pl
from jax.experimental.pallas import tpu as pltpu

_K = 31
_P = _K // 2
_EPS = 1e-05
_N = 1024


def _band_matrix():
    i = jax.lax.broadcasted_iota(jnp.int32, (_N, _N), 0)
    j = jax.lax.broadcasted_iota(jnp.int32, (_N, _N), 1)
    return (jnp.abs(i - j) <= _P).astype(jnp.bfloat16)


def _lcn_kernel(b_ref, x_ref, o_ref):
    x = x_ref[0]
    band = b_ref[...]
    inv = jnp.float32(1.0 / (_K * _K))

    xb = x.astype(jnp.bfloat16)
    v = jnp.dot(band, xb, preferred_element_type=jnp.float32)
    mean = jnp.dot(v.astype(jnp.bfloat16), band,
                   preferred_element_type=jnp.float32) * inv

    x2b = (x * x).astype(jnp.bfloat16)
    v2 = jnp.dot(band, x2b, preferred_element_type=jnp.float32)
    sq_mean = jnp.dot(v2.astype(jnp.bfloat16), band,
                      preferred_element_type=jnp.float32) * inv

    var = sq_mean - mean * mean
    std = jnp.sqrt(jnp.maximum(var, _EPS))
    normalized = (x - mean) / (std + _EPS)
    o_ref[0] = jax.nn.sigmoid(normalized * 0.5)


@functools.partial(jax.jit, static_argnames=("interpret",))
def kernel(x, interpret=False):
    b, c, h, w = x.shape
    xr = x.reshape(b * c, h, w)
    band = _band_matrix()
    out = pl.pallas_call(
        _lcn_kernel,
        out_shape=jax.ShapeDtypeStruct(xr.shape, xr.dtype),
        grid=(b * c,),
        in_specs=[
            pl.BlockSpec((_N, _N), lambda i: (0, 0)),
            pl.BlockSpec((1, _N, _N), lambda i: (i, 0, 0)),
        ],
        out_specs=pl.BlockSpec((1, _N, _N), lambda i: (i, 0, 0)),
        compiler_params=pltpu.CompilerParams(
            dimension_semantics=("parallel",),
            vmem_limit_bytes=56 * 1024 * 1024,
        ),
        name="lcn_fused",
        interpret=interpret,
    )(band, xr)
    return out.reshape(b, c, h, w)


# speedup vs baseline: 292.7680x; 1.6553x over previous
import functools

import jax
import jax.numpy as jnp
from jax.experimental import pallas as pl
from jax.experimental.pallas import tpu as pltpu

_K = 31
_P = _K // 2
_EPS = 1e-05
_N = 1024
_BLK = 256
_W = 512

_LOS = (0, 128, 384, 512)
_DS = (0, 128, 128, 256)
_PIDX = (0, 1, 1, 2)
_DVALS = (0, 128, 256)


def _band_patterns():
    m = jax.lax.broadcasted_iota(jnp.int32, (3, _BLK, _W), 1)
    k = jax.lax.broadcasted_iota(jnp.int32, (3, _BLK, _W), 2)
    d = jnp.asarray(_DVALS, jnp.int32).reshape(3, 1, 1)
    p = (jnp.abs(m + d - k) <= _P).astype(jnp.bfloat16)
    return p, jnp.transpose(p, (0, 2, 1))


def _lcn_kernel(p_ref, q_ref, x_ref, o_ref, xb2_ref, vb2_ref):
    x = x_ref[0]
    inv = jnp.float32(1.0 / (_K * _K))

    xb2_ref[:, :_N] = x.astype(jnp.bfloat16)
    xb2_ref[:, _N:] = (x * x).astype(jnp.bfloat16)

    for b in range(4):
        lo, pi = _LOS[b], _PIDX[b]
        vb2_ref[b * _BLK:(b + 1) * _BLK, :] = jnp.dot(
            p_ref[pi], xb2_ref[lo:lo + _W, :],
            preferred_element_type=jnp.float32).astype(jnp.bfloat16)

    for b in range(4):
        lo, pi = _LOS[b], _PIDX[b]
        q = q_ref[pi]
        mean = jnp.dot(vb2_ref[:, lo:lo + _W], q,
                       preferred_element_type=jnp.float32) * inv
        sq_mean = jnp.dot(vb2_ref[:, _N + lo:_N + lo + _W], q,
                          preferred_element_type=jnp.float32) * inv
        var = sq_mean - mean * mean
        std = jnp.sqrt(jnp.maximum(var, _EPS))
        xs = x[:, b * _BLK:(b + 1) * _BLK]
        normalized = (xs - mean) / (std + _EPS)
        o_ref[0, :, b * _BLK:(b + 1) * _BLK] = jax.nn.sigmoid(
            normalized * 0.5)


@functools.partial(jax.jit, static_argnames=("interpret",))
def kernel(x, interpret=False):
    b, c, h, w = x.shape
    xr = x.reshape(b * c, h, w)
    p, q = _band_patterns()
    out = pl.pallas_call(
        _lcn_kernel,
        out_shape=jax.ShapeDtypeStruct(xr.shape, xr.dtype),
        grid=(b * c,),
        in_specs=[
            pl.BlockSpec((3, _BLK, _W), lambda i: (0, 0, 0)),
            pl.BlockSpec((3, _W, _BLK), lambda i: (0, 0, 0)),
            pl.BlockSpec((1, _N, _N), lambda i: (i, 0, 0)),
        ],
        out_specs=pl.BlockSpec((1, _N, _N), lambda i: (i, 0, 0)),
        scratch_shapes=[
            pltpu.VMEM((_N, 2 * _N), jnp.bfloat16),
            pltpu.VMEM((_N, 2 * _N), jnp.bfloat16),
        ],
        compiler_params=pltpu.CompilerParams(
            dimension_semantics=("parallel",),
            vmem_limit_bytes=56 * 1024 * 1024,
        ),
        name="lcn_fused",
        interpret=interpret,
    )(p, q, xr)
    return out.reshape(b, c, h, w)


# speedup vs baseline: 407.2480x; 1.3910x over previous
import functools

import jax
import jax.numpy as jnp
from jax.experimental import pallas as pl
from jax.experimental.pallas import tpu as pltpu

_K = 31
_P = _K // 2
_EPS = 1e-05
_N = 1024

_VB, _VW = 128, 256
_VLOS = (0, 64, 192, 320, 448, 576, 704, 768)
_VPIDX = (0, 1, 1, 1, 1, 1, 1, 2)
_VDS = (0, 64, 128)
_HB, _HW = 256, 512
_HLOS = (0, 128, 384, 512)
_HPIDX = (0, 1, 1, 2)
_HDS = (0, 128, 256)


def _band_patterns():
    m = jax.lax.broadcasted_iota(jnp.int32, (3, _VB, _VW), 1)
    k = jax.lax.broadcasted_iota(jnp.int32, (3, _VB, _VW), 2)
    d = jnp.asarray(_VDS, jnp.int32).reshape(3, 1, 1)
    pv = (jnp.abs(m + d - k) <= _P).astype(jnp.bfloat16)
    kk = jax.lax.broadcasted_iota(jnp.int32, (3, _HW, _HB), 1)
    n = jax.lax.broadcasted_iota(jnp.int32, (3, _HW, _HB), 2)
    dh = jnp.asarray(_HDS, jnp.int32).reshape(3, 1, 1)
    qh = (jnp.abs(n + dh - kk) <= _P).astype(jnp.bfloat16)
    return pv, qh


def _lcn_kernel(pv_ref, qh_ref, x_ref, o_ref, xb2_ref, vb2_ref):
    x = x_ref[0]
    inv = jnp.float32(1.0 / (_K * _K))

    xb = x.astype(jnp.bfloat16)
    xb2_ref[:, :_N] = xb
    xb2_ref[:, _N:] = xb * xb

    for b in range(8):
        lo, pi = _VLOS[b], _VPIDX[b]
        vb2_ref[b * _VB:(b + 1) * _VB, :] = jnp.dot(
            pv_ref[pi], xb2_ref[lo:lo + _VW, :],
            preferred_element_type=jnp.float32).astype(jnp.bfloat16)

    for b in range(4):
        lo, pi = _HLOS[b], _HPIDX[b]
        q = qh_ref[pi]
        mean = jnp.dot(vb2_ref[:, lo:lo + _HW], q,
                       preferred_element_type=jnp.float32) * inv
        sq_mean = jnp.dot(vb2_ref[:, _N + lo:_N + lo + _HW], q,
                          preferred_element_type=jnp.float32) * inv
        var = sq_mean - mean * mean
        rs = jax.lax.rsqrt(jnp.maximum(var, _EPS))
        inv_d = rs * (1.0 - _EPS * rs)
        xs = x[:, b * _HB:(b + 1) * _HB]
        normalized = (xs - mean) * inv_d
        o_ref[0, :, b * _HB:(b + 1) * _HB] = (
            0.5 * jnp.tanh(normalized * 0.25) + 0.5)


@functools.partial(jax.jit, static_argnames=("interpret",))
def kernel(x, interpret=False):
    b, c, h, w = x.shape
    xr = x.reshape(b * c, h, w)
    pv, qh = _band_patterns()
    out = pl.pallas_call(
        _lcn_kernel,
        out_shape=jax.ShapeDtypeStruct(xr.shape, xr.dtype),
        grid=(b * c,),
        in_specs=[
            pl.BlockSpec((3, _VB, _VW), lambda i: (0, 0, 0)),
            pl.BlockSpec((3, _HW, _HB), lambda i: (0, 0, 0)),
            pl.BlockSpec((1, _N, _N), lambda i: (i, 0, 0)),
        ],
        out_specs=pl.BlockSpec((1, _N, _N), lambda i: (i, 0, 0)),
        scratch_shapes=[
            pltpu.VMEM((_N, 2 * _N), jnp.bfloat16),
            pltpu.VMEM((_N, 2 * _N), jnp.bfloat16),
        ],
        compiler_params=pltpu.CompilerParams(
            dimension_semantics=("parallel",),
            vmem_limit_bytes=56 * 1024 * 1024,
        ),
        name="lcn_fused",
        interpret=interpret,
    )(pv, qh, xr)
    return out.reshape(b, c, h, w)
